# baseline (speedup 1.0000x reference)
import functools
import math

import jax
import jax.numpy as jnp
from jax import lax
from jax.experimental import pallas as pl
from jax.experimental.pallas import tpu as pltpu
from jax.experimental.pallas import tpu_sc as plsc

VOCAB = 100000
D = 768
B = 4
S = 4096
N = B * S
SCALE = math.sqrt(float(D))

_info = plsc.get_sparse_core_info()
NC = _info.num_cores
NS = _info.num_subcores
NW = NC * NS
S_W = S // NW
R = 32
NCH = S_W // R
LANES = 16
JV = D // LANES


def _sc_embed(idx_arr, table, pe_s):
    mesh = plsc.VectorSubcoreMesh(core_axis_name="c", subcore_axis_name="s")

    @functools.partial(
        pl.kernel,
        mesh=mesh,
        out_type=jax.ShapeDtypeStruct((N, D), jnp.float32),
        scratch_types=[
            pltpu.VMEM((NCH * B, R), jnp.int32),
            pltpu.VMEM((B, R, D), jnp.float32),
            pltpu.VMEM((R, D), jnp.float32),
            pltpu.SemaphoreType.DMA((B,)),
            pltpu.SemaphoreType.DMA((B,)),
        ],
    )
    def k(idx_hbm, table_hbm, pe_hbm, out_hbm,
          idx_v, rows_v, pe_v, g_sem, o_sem):
        wid = lax.axis_index("s") * NC + lax.axis_index("c")
        sbase = wid * S_W

        def drain_out(b):
            pltpu.make_async_copy(
                rows_v.at[b], out_hbm.at[pl.ds(0, R)], o_sem.at[b]).wait()

        pltpu.sync_copy(idx_hbm.at[wid], idx_v)

        def chunk(sc, _):
            for b in range(B):
                pl.when(sc >= 1)(lambda b=b: drain_out(b))
            gh = [
                pltpu.async_copy(
                    table_hbm.at[idx_v.at[sc * B + b]],
                    rows_v.at[b], g_sem.at[b])
                for b in range(B)
            ]
            pltpu.sync_copy(pe_hbm.at[pl.ds(sbase + sc * R, R)], pe_v)
            for b in range(B):
                gh[b].wait()

            def row(r, _):
                for j in range(JV):
                    sl = pl.ds(j * LANES, LANES)
                    pe_reg = pe_v[r, sl]
                    for b in range(B):
                        rows_v[b, r, sl] = rows_v[b, r, sl] * SCALE + pe_reg
                return 0

            lax.fori_loop(0, R, row, 0)
            for b in range(B):
                pltpu.async_copy(
                    rows_v.at[b],
                    out_hbm.at[pl.ds(b * S + sbase + sc * R, R)],
                    o_sem.at[b])
            return 0

        lax.fori_loop(0, NCH, chunk, 0)
        for b in range(B):
            drain_out(b)

    return k(idx_arr, table, pe_s)


def kernel(x, table, pe):
    idx_arr = (x.reshape(B, NW, NCH, R)
                .transpose(1, 2, 0, 3)
                .reshape(NW, NCH * B, R))
    out = _sc_embed(idx_arr, table, pe[:S])
    return out.reshape(B, S, D)
